# baseline (speedup 1.0000x reference)
import jax
import jax.numpy as jnp
from jax.experimental import pallas as pl


def _bias_add_kernel(x_ref, b_ref, o_ref):
    o_ref[...] = x_ref[...] + b_ref[...]


def kernel(x, edge_index, edge_attr, n_w1, n_b1, n_w2, n_b2, e_w1, e_b1, e_w2, e_b2, gat_lin, gat_lin_edge, att_src, att_dst, att_edge, gat_bias, eu_w, eu_b):
    h = jax.nn.relu(x @ n_w1 + n_b1)
    h = jax.nn.relu(h @ n_w2 + n_b2)
    ea = jax.nn.relu(edge_attr @ e_w1 + e_b1)
    ea = jax.nn.relu(ea @ e_w2 + e_b2)
    n = h.shape[0]
    src = edge_index[0]
    dst = edge_index[1]
    ones = jnp.ones((src.shape[0],), dtype=h.dtype)
    deg = jax.ops.segment_sum(ones, dst, num_segments=n)
    loop_attr = jax.ops.segment_sum(ea, dst, num_segments=n) / jnp.clip(deg, 1.0)[:, None]
    ar = jnp.arange(n, dtype=src.dtype)
    src_a = jnp.concatenate([src, ar])
    dst_a = jnp.concatenate([dst, ar])
    ea_a = jnp.concatenate([ea, loop_attr], axis=0)
    xw = h @ gat_lin
    a_src = xw @ att_src
    a_dst = xw @ att_dst
    e_proj = ea_a @ gat_lin_edge
    a_e = e_proj @ att_edge
    alpha = jax.nn.leaky_relu(a_src[src_a] + a_dst[dst_a] + a_e, 0.2)
    amax = jax.ops.segment_max(alpha, dst_a, num_segments=n)
    alpha = jnp.exp(alpha - amax[dst_a])
    denom = jax.ops.segment_sum(alpha, dst_a, num_segments=n)
    alpha = alpha / (denom[dst_a] + 1e-16)
    out0 = jax.ops.segment_sum(xw[src_a] * alpha[:, None], dst_a, num_segments=n)
    out = pl.pallas_call(
        _bias_add_kernel,
        out_shape=jax.ShapeDtypeStruct(out0.shape, out0.dtype),
    )(out0, jnp.broadcast_to(gat_bias, out0.shape))
    new_ea = jnp.concatenate([out[src], out[dst], ea], axis=1) @ eu_w + eu_b
    return (out, new_ea)


# speedup vs baseline: 4.3174x; 4.3174x over previous
import functools

import jax
import jax.numpy as jnp
from jax import lax
from jax.experimental import pallas as pl
from jax.experimental.pallas import tpu as pltpu
from jax.experimental.pallas import tpu_sc as plsc

N = 10000
E = 320000
H = 128
DE = 16

NP = 10240
NW = 32
EPW = 10240
EP = NW * EPW
NCHUNK = 80
G = 128
NB = NP // 128
SLICE = NP // 16

_f32 = jnp.float32
_i32 = jnp.int32


def _node_body(x_ref, w1_ref, b1_ref, w2_ref, b2_ref, gl_ref, as_ref, ad_ref,
               xw_ref, asrc_ref, adst_ref, ms_ref, md_ref):
    xb = x_ref[...]
    h = jnp.maximum(jnp.dot(xb, w1_ref[...], preferred_element_type=_f32) + b1_ref[...], 0.0)
    h = jnp.maximum(jnp.dot(h, w2_ref[...], preferred_element_type=_f32) + b2_ref[...], 0.0)
    xw = jnp.dot(h, gl_ref[...], preferred_element_type=_f32)
    asr = jnp.dot(xw, as_ref[...], preferred_element_type=_f32)
    adt = jnp.dot(xw, ad_ref[...], preferred_element_type=_f32)
    xw_ref[...] = xw
    asrc_ref[...] = asr
    adst_ref[...] = adt
    bs = jnp.max(asr, axis=(0, 1), keepdims=True)
    bd = jnp.max(adt, axis=(0, 1), keepdims=True)

    @pl.when(pl.program_id(0) == 0)
    def _():
        ms_ref[...] = bs
        md_ref[...] = bd

    @pl.when(pl.program_id(0) != 0)
    def _():
        ms_ref[...] = jnp.maximum(ms_ref[...], bs)
        md_ref[...] = jnp.maximum(md_ref[...], bd)


def _node_pass(x_p, n_w1, n_b1, n_w2, n_b2, gat_lin, att_src, att_dst):
    full = lambda s: pl.BlockSpec(s, lambda i: (0, 0))
    return pl.pallas_call(
        _node_body,
        grid=(NB,),
        in_specs=[
            pl.BlockSpec((128, H), lambda i: (i, 0)),
            full((H, H)), full((1, H)), full((H, H)), full((1, H)),
            full((H, H)), full((H, 1)), full((H, 1)),
        ],
        out_specs=[
            pl.BlockSpec((128, H), lambda i: (i, 0)),
            pl.BlockSpec((128, 1), lambda i: (i, 0)),
            pl.BlockSpec((128, 1), lambda i: (i, 0)),
            full((1, 1)), full((1, 1)),
        ],
        out_shape=[
            jax.ShapeDtypeStruct((NP, H), _f32),
            jax.ShapeDtypeStruct((NP, 1), _f32),
            jax.ShapeDtypeStruct((NP, 1), _f32),
            jax.ShapeDtypeStruct((1, 1), _f32),
            jax.ShapeDtypeStruct((1, 1), _f32),
        ],
    )(x_p, n_w1, n_b1.reshape(1, H), n_w2, n_b2.reshape(1, H),
      gat_lin, att_src.reshape(H, 1), att_dst.reshape(H, 1))


def _edge_body(eb_ref, w1_ref, b1_ref, w2_ref, b2_ref, gle_ref, ae_ref,
               ea_ref, s_ref, sm_ref):
    eb = eb_ref[...]
    e1 = jnp.maximum(jnp.dot(eb, w1_ref[...], preferred_element_type=_f32) + b1_ref[...], 0.0)
    e2 = jnp.maximum(jnp.dot(e1, w2_ref[...], preferred_element_type=_f32) + b2_ref[...], 0.0)
    v = jnp.dot(gle_ref[...], ae_ref[...], preferred_element_type=_f32)
    s = jnp.dot(e2, v, preferred_element_type=_f32)
    ea_ref[...] = e2
    s_ref[...] = s
    bm = jnp.max(s, axis=(0, 1), keepdims=True)

    @pl.when(pl.program_id(0) == 0)
    def _():
        sm_ref[...] = bm

    @pl.when(pl.program_id(0) != 0)
    def _():
        sm_ref[...] = jnp.maximum(sm_ref[...], bm)


def _edge_pass(ea_p, e_w1, e_b1, e_w2, e_b2, gat_lin_edge, att_edge):
    full = lambda s: pl.BlockSpec(s, lambda i: (0, 0))
    nblk = EP // 1024
    return pl.pallas_call(
        _edge_body,
        grid=(nblk,),
        in_specs=[
            pl.BlockSpec((1024, DE), lambda i: (i, 0)),
            full((DE, DE)), full((1, DE)), full((DE, DE)), full((1, DE)),
            full((DE, H)), full((H, 1)),
        ],
        out_specs=[
            pl.BlockSpec((1024, DE), lambda i: (i, 0)),
            pl.BlockSpec((1024, 1), lambda i: (i, 0)),
            full((1, 1)),
        ],
        out_shape=[
            jax.ShapeDtypeStruct((EP, DE), _f32),
            jax.ShapeDtypeStruct((EP, 1), _f32),
            jax.ShapeDtypeStruct((1, 1), _f32),
        ],
    )(ea_p, e_w1, e_b1.reshape(1, DE), e_w2, e_b2.reshape(1, DE),
      gat_lin_edge, att_edge.reshape(H, 1))


def _mid1_body(d0_ref, d1_ref, s0_ref, s1_ref, as_ref, ad_ref,
               ms_ref, md_ref, sm_ref, m_ref, wl_ref, sl_ref):
    deg = d0_ref[...] + d1_ref[...]
    ssum = s0_ref[...] + s1_ref[...]
    sloop = ssum / jnp.maximum(deg, 1.0)
    asr = as_ref[...]
    adt = ad_ref[...]
    tb = jnp.maximum(ms_ref[...] + sm_ref[...], asr + sloop) + adt
    m = jnp.where(tb >= 0, tb, 0.2 * tb)
    tl = asr + adt + sloop
    aloop = jnp.where(tl >= 0, tl, 0.2 * tl)
    m_ref[...] = m
    wl_ref[...] = jnp.exp(aloop - m)
    sl_ref[...] = sloop


def _mid1(deg_p, ssum_p, a_src, a_dst, ms, md, sm):
    full = lambda s: pl.BlockSpec(s, lambda i: (0, 0))
    col = lambda: pl.BlockSpec((128, 1), lambda i: (i, 0))
    d0 = deg_p[0].reshape(NP, 1)
    d1 = deg_p[1].reshape(NP, 1)
    s0 = ssum_p[0].reshape(NP, 1)
    s1 = ssum_p[1].reshape(NP, 1)
    return pl.pallas_call(
        _mid1_body,
        grid=(NB,),
        in_specs=[col(), col(), col(), col(), col(), col(),
                  full((1, 1)), full((1, 1)), full((1, 1))],
        out_specs=[col(), col(), col()],
        out_shape=[
            jax.ShapeDtypeStruct((NP, 1), _f32),
            jax.ShapeDtypeStruct((NP, 1), _f32),
            jax.ShapeDtypeStruct((NP, 1), _f32),
        ],
    )(d0, d1, s0, s1, a_src, a_dst, ms, md, sm)


def _mid2_body(m0_ref, m1_ref, xw_ref, wl_ref, dn0_ref, dn1_ref, gb_ref,
               w1_ref, w2_ref, out_ref, o1_ref, o2_ref):
    wl = wl_ref[...]
    den = dn0_ref[...] + dn1_ref[...] + wl + 1e-16
    out = (m0_ref[...] + m1_ref[...] + wl * xw_ref[...]) / den + gb_ref[...]
    out_ref[...] = out
    o1_ref[...] = jnp.dot(out, w1_ref[...], preferred_element_type=_f32)
    o2_ref[...] = jnp.dot(out, w2_ref[...], preferred_element_type=_f32)


def _mid2(msg_p, den_p, xw, wloop, gat_bias, eu_w1, eu_w2):
    full = lambda s: pl.BlockSpec(s, lambda i: (0, 0))
    row = lambda: pl.BlockSpec((128, H), lambda i: (i, 0))
    col = lambda: pl.BlockSpec((128, 1), lambda i: (i, 0))
    return pl.pallas_call(
        _mid2_body,
        grid=(NB,),
        in_specs=[row(), row(), row(), col(), col(), col(), full((1, H)),
                  full((H, H)), full((H, H))],
        out_specs=[row(), row(), row()],
        out_shape=[
            jax.ShapeDtypeStruct((NP, H), _f32),
            jax.ShapeDtypeStruct((NP, H), _f32),
            jax.ShapeDtypeStruct((NP, H), _f32),
        ],
    )(msg_p[0], msg_p[1], xw, wloop, den_p[0].reshape(NP, 1),
      den_p[1].reshape(NP, 1), gat_bias.reshape(1, H), eu_w1, eu_w2)


def _final_body(g_ref, ea_ref, w3_ref, b_ref, o_ref):
    o_ref[...] = (g_ref[...] +
                  jnp.dot(ea_ref[...], w3_ref[...], preferred_element_type=_f32) +
                  b_ref[...])


def _final(g, ea, eu_w3, eu_b):
    full = lambda s: pl.BlockSpec(s, lambda i: (0, 0))
    nblk = E // 512
    return pl.pallas_call(
        _final_body,
        grid=(nblk,),
        in_specs=[
            pl.BlockSpec((512, H), lambda i: (i, 0)),
            pl.BlockSpec((512, DE), lambda i: (i, 0)),
            full((DE, H)), full((1, H)),
        ],
        out_specs=pl.BlockSpec((512, H), lambda i: (i, 0)),
        out_shape=jax.ShapeDtypeStruct((E, H), _f32),
    )(g, ea, eu_w3, eu_b.reshape(1, H))


_MESH = plsc.VectorSubcoreMesh(core_axis_name="c", subcore_axis_name="s")


def _wid():
    return lax.axis_index("s") * 2 + lax.axis_index("c")


def _zero_1d(ref, nvec):
    z = jnp.zeros((16,), _f32)

    def body(i, _):
        ref[pl.ds(i * 16, 16)] = z
        return ()

    lax.fori_loop(0, nvec, body, ())


def _sc_a1_body(dst_hbm, s_hbm, deg_out, ssum_out,
                dstv, sv, onesv, zb, deg_sp, ssum_sp, sem):
    cid = lax.axis_index("c")
    sid = lax.axis_index("s")
    wid = _wid()
    _zero_1d(zb, SLICE // 16)
    pltpu.sync_copy(zb, deg_sp.at[pl.ds(sid * SLICE, SLICE)])
    pltpu.sync_copy(zb, ssum_sp.at[pl.ds(sid * SLICE, SLICE)])
    plsc.subcore_barrier()
    pltpu.sync_copy(dst_hbm.at[wid], dstv)
    pltpu.sync_copy(s_hbm.at[wid], sv)
    lane = jax.lax.iota(_i32, 16)

    def ones_body(c, _):
        base = wid * EPW + c * G
        for g in range(8):
            eid = base + g * 16 + lane
            onesv[c, pl.ds(g * 16, 16)] = jnp.where(eid < E, 1.0, 0.0)
        return ()

    lax.fori_loop(0, NCHUNK, ones_body, ())

    def scat_body(c, _):
        pltpu.sync_copy(onesv.at[c], deg_sp.at[dstv.at[c]], add=True)
        pltpu.sync_copy(sv.at[c], ssum_sp.at[dstv.at[c]], add=True)
        return ()

    lax.fori_loop(0, NCHUNK, scat_body, ())
    plsc.subcore_barrier()
    pltpu.sync_copy(deg_sp.at[pl.ds(sid * SLICE, SLICE)],
                    deg_out.at[cid, pl.ds(sid * SLICE, SLICE)])
    pltpu.sync_copy(ssum_sp.at[pl.ds(sid * SLICE, SLICE)],
                    ssum_out.at[cid, pl.ds(sid * SLICE, SLICE)])


def _sc_a1(dst3, s3):
    return pl.kernel(
        _sc_a1_body,
        out_type=[
            jax.ShapeDtypeStruct((2, NP), _f32),
            jax.ShapeDtypeStruct((2, NP), _f32),
        ],
        mesh=_MESH,
        scratch_types=[
            pltpu.VMEM((NCHUNK, G), _i32),
            pltpu.VMEM((NCHUNK, G), _f32),
            pltpu.VMEM((NCHUNK, G), _f32),
            pltpu.VMEM((SLICE,), _f32),
            pltpu.VMEM_SHARED((NP,), _f32),
            pltpu.VMEM_SHARED((NP,), _f32),
            pltpu.SemaphoreType.DMA,
        ],
    )(dst3, s3)


NPH = NP // 2
PSLICE = NPH // 16


def _sc_a2_body(src_hbm, dst_hbm, s_hbm, asrc_hbm, adst_hbm, m_hbm, xw_hbm,
                msg_out, den_out,
                srcv, dstv, sv, wtile, zb, idxb,
                rows0, ag0, bg0, mg0,
                rows1, ag1, bg1, mg1,
                msg_sp, den_sp, gsem0, gsem1, ssem, dsem):
    cid = lax.axis_index("c")
    sid = lax.axis_index("s")
    wid = _wid()
    bufs = ((rows0, ag0, bg0, mg0, gsem0), (rows1, ag1, bg1, mg1, gsem1))
    z = jnp.zeros((16,), _f32)
    lane = jax.lax.iota(_i32, 16)

    def zero_msg_slice():
        def zrow(i, _):
            for v in range(8):
                rows0[i, pl.ds(v * 16, 16)] = z
            return ()

        lax.fori_loop(0, G, zrow, ())
        for k in range(PSLICE // G):
            pltpu.sync_copy(rows0, msg_sp.at[pl.ds(sid * PSLICE + k * G, G)])
        pltpu.sync_copy(rows0.at[pl.ds(0, PSLICE % G)],
                        msg_sp.at[pl.ds(sid * PSLICE + (PSLICE // G) * G,
                                        PSLICE % G)])

    zero_msg_slice()
    _zero_1d(zb, SLICE // 16)
    pltpu.sync_copy(zb, den_sp.at[pl.ds(sid * SLICE, SLICE)])
    pltpu.sync_copy(src_hbm.at[wid], srcv)
    pltpu.sync_copy(dst_hbm.at[wid], dstv)
    pltpu.sync_copy(s_hbm.at[wid], sv)
    plsc.subcore_barrier()

    def fire(c, buf, ph):
        rws, ag, bg, mg, sem = buf
        pltpu.async_copy(xw_hbm.at[srcv.at[c]], rws, sem)
        if ph == 0:
            pltpu.async_copy(asrc_hbm.at[srcv.at[c]], ag, sem)
            pltpu.async_copy(adst_hbm.at[dstv.at[c]], bg, sem)
            pltpu.async_copy(m_hbm.at[dstv.at[c]], mg, sem)

    def drain(c, buf, ph):
        rws, ag, bg, mg, sem = buf
        pltpu.make_async_copy(xw_hbm.at[srcv.at[c]], rws, sem).wait()
        if ph == 0:
            pltpu.make_async_copy(asrc_hbm.at[srcv.at[c]], ag, sem).wait()
            pltpu.make_async_copy(adst_hbm.at[dstv.at[c]], bg, sem).wait()
            pltpu.make_async_copy(m_hbm.at[dstv.at[c]], mg, sem).wait()

    def make_phase(ph):
        nbase = ph * NPH

        def outer(i, _):
            for b in (0, 1):
                c = 2 * i + b
                buf = bufs[b]
                rws, ag, bg, mg, _sem = buf
                drain(c, buf, ph)

                @pl.when(c + 1 < NCHUNK)
                def _():
                    fire(c + 1, bufs[1 - b], ph)

                def group(g, _):
                    d16 = dstv[c, pl.ds(g * 16, 16)]
                    if ph == 0:
                        s16 = sv[c, pl.ds(g * 16, 16)]
                        av = ag[pl.ds(g * 16, 16)]
                        bv = bg[pl.ds(g * 16, 16)]
                        mv = mg[pl.ds(g * 16, 16)]
                        t = av + bv + s16
                        al = jnp.where(t >= 0, t, 0.2 * t)
                        eid = wid * EPW + c * G + g * 16 + lane
                        w16 = jnp.where(eid < E, jnp.exp(al - mv), 0.0)
                        wtile[c, pl.ds(g * 16, 16)] = w16
                    else:
                        w16 = wtile[c, pl.ds(g * 16, 16)]
                    dloc = d16 - nbase
                    inrange = (dloc >= 0) & (dloc < NPH)
                    idxb[pl.ds(g * 16, 16)] = jnp.where(inrange, dloc, NPH)
                    for e in range(16):
                        wv = w16.at[jnp.full((16,), e, _i32)].get(
                            mode="promise_in_bounds")
                        row = g * 16 + e
                        for v in range(8):
                            rws[row, pl.ds(v * 16, 16)] = (
                                rws[row, pl.ds(v * 16, 16)] * wv)
                    return ()

                lax.fori_loop(0, 8, group, ())
                if ph == 0:
                    pltpu.async_copy(wtile.at[c], den_sp.at[dstv.at[c]],
                                     dsem, add=True)
                pltpu.async_copy(rws, msg_sp.at[idxb], ssem, add=True).wait()
                if ph == 0:
                    pltpu.make_async_copy(wtile.at[c], den_sp.at[dstv.at[c]],
                                          dsem).wait()
            return ()

        fire(0, bufs[0], ph)
        lax.fori_loop(0, NCHUNK // 2, outer, ())
        plsc.subcore_barrier()
        pltpu.sync_copy(msg_sp.at[pl.ds(sid * PSLICE, PSLICE)],
                        msg_out.at[cid, pl.ds(nbase + sid * PSLICE, PSLICE)])

    make_phase(0)
    pltpu.sync_copy(den_sp.at[pl.ds(sid * SLICE, SLICE)],
                    den_out.at[cid, pl.ds(sid * SLICE, SLICE)])
    zero_msg_slice()
    plsc.subcore_barrier()
    make_phase(1)


def _sc_a2(src3, dst3, s3, a_src, a_dst, m_col, xw):
    return pl.kernel(
        _sc_a2_body,
        out_type=[
            jax.ShapeDtypeStruct((2, NP, H), _f32),
            jax.ShapeDtypeStruct((2, NP), _f32),
        ],
        mesh=_MESH,
        scratch_types=[
            pltpu.VMEM((NCHUNK, G), _i32),
            pltpu.VMEM((NCHUNK, G), _i32),
            pltpu.VMEM((NCHUNK, G), _f32),
            pltpu.VMEM((NCHUNK, G), _f32),
            pltpu.VMEM((SLICE,), _f32),
            pltpu.VMEM((G,), _i32),
            pltpu.VMEM((G, H), _f32),
            pltpu.VMEM((G,), _f32),
            pltpu.VMEM((G,), _f32),
            pltpu.VMEM((G,), _f32),
            pltpu.VMEM((G, H), _f32),
            pltpu.VMEM((G,), _f32),
            pltpu.VMEM((G,), _f32),
            pltpu.VMEM((G,), _f32),
            pltpu.VMEM_SHARED((NPH + 8, H), _f32),
            pltpu.VMEM_SHARED((NP,), _f32),
            pltpu.SemaphoreType.DMA,
            pltpu.SemaphoreType.DMA,
            pltpu.SemaphoreType.DMA,
            pltpu.SemaphoreType.DMA,
        ],
    )(src3, dst3, s3, a_src.reshape(NP), a_dst.reshape(NP), m_col.reshape(NP),
      xw)


def _sc_b_body(src_hbm, dst_hbm, o1_hbm, o2_hbm, g_out,
               srcv, dstv, r1a, r2a, r1b, r2b,
               gsem0, gsem1, wsem0, wsem1):
    wid = _wid()
    pltpu.sync_copy(src_hbm.at[wid], srcv)
    pltpu.sync_copy(dst_hbm.at[wid], dstv)
    bufs = ((r1a, r2a, gsem0, wsem0), (r1b, r2b, gsem1, wsem1))

    def out_slice(c):
        return g_out.at[pl.ds(wid * EPW + c * G, G)]

    def fire(c, buf):
        r1, r2, gsem, _w = buf
        pltpu.async_copy(o1_hbm.at[srcv.at[c]], r1, gsem)
        pltpu.async_copy(o2_hbm.at[dstv.at[c]], r2, gsem)

    def drain(c, buf):
        r1, r2, gsem, _w = buf
        pltpu.make_async_copy(o1_hbm.at[srcv.at[c]], r1, gsem).wait()
        pltpu.make_async_copy(o2_hbm.at[dstv.at[c]], r2, gsem).wait()

    fire(0, bufs[0])

    def outer(i, _):
        for b in (0, 1):
            c = 2 * i + b
            r1, r2, _g, wsem = bufs[b]

            @pl.when(c >= 2)
            def _():
                pltpu.make_async_copy(r1, out_slice(c - 2), wsem).wait()

            drain(c, bufs[b])

            @pl.when(c + 1 < NCHUNK)
            def _():
                fire(c + 1, bufs[1 - b])

            def row_add(r, _):
                for v in range(8):
                    r1[r, pl.ds(v * 16, 16)] = (
                        r1[r, pl.ds(v * 16, 16)] + r2[r, pl.ds(v * 16, 16)])
                return ()

            lax.fori_loop(0, G, row_add, ())
            pltpu.async_copy(r1, out_slice(c), wsem)
        return ()

    lax.fori_loop(0, NCHUNK // 2, outer, ())
    for last in (NCHUNK - 2, NCHUNK - 1):
        r1, _r2, _g, wsem = bufs[last % 2]
        pltpu.make_async_copy(r1, out_slice(last), wsem).wait()


def _sc_b(src3, dst3, out1, out2):
    return pl.kernel(
        _sc_b_body,
        out_type=jax.ShapeDtypeStruct((EP, H), _f32),
        mesh=_MESH,
        scratch_types=[
            pltpu.VMEM((NCHUNK, G), _i32),
            pltpu.VMEM((NCHUNK, G), _i32),
            pltpu.VMEM((G, H), _f32),
            pltpu.VMEM((G, H), _f32),
            pltpu.VMEM((G, H), _f32),
            pltpu.VMEM((G, H), _f32),
            pltpu.SemaphoreType.DMA,
            pltpu.SemaphoreType.DMA,
            pltpu.SemaphoreType.DMA,
            pltpu.SemaphoreType.DMA,
        ],
    )(src3, dst3, out1, out2)


def kernel(x, edge_index, edge_attr, n_w1, n_b1, n_w2, n_b2, e_w1, e_b1,
           e_w2, e_b2, gat_lin, gat_lin_edge, att_src, att_dst, att_edge,
           gat_bias, eu_w, eu_b):
    x_p = jnp.pad(x, ((0, NP - N), (0, 0)))
    ea_p = jnp.pad(edge_attr, ((0, EP - E), (0, 0)))
    src = edge_index[0].astype(_i32)
    dst = edge_index[1].astype(_i32)
    src3 = jnp.pad(src, (0, EP - E)).reshape(NW, NCHUNK, G)
    dst3 = jnp.pad(dst, (0, EP - E)).reshape(NW, NCHUNK, G)

    xw, a_src, a_dst, ms, md = _node_pass(
        x_p, n_w1, n_b1, n_w2, n_b2, gat_lin, att_src, att_dst)
    ea, s_col, sm = _edge_pass(
        ea_p, e_w1, e_b1, e_w2, e_b2, gat_lin_edge, att_edge)
    s3 = s_col.reshape(NW, NCHUNK, G)

    deg_p, ssum_p = _sc_a1(dst3, s3)
    m_col, wloop, _sloop = _mid1(deg_p, ssum_p, a_src, a_dst, ms, md, sm)
    msg_p, den_p = _sc_a2(src3, dst3, s3, a_src, a_dst, m_col, xw)
    out, out1, out2 = _mid2(msg_p, den_p, xw, wloop, gat_bias,
                            eu_w[:H], eu_w[H:2 * H])
    g = _sc_b(src3, dst3, out1, out2)
    new_ea = _final(g[:E], ea[:E], eu_w[2 * H:], eu_b)
    return (out[:N], new_ea)


# speedup vs baseline: 4.3177x; 1.0001x over previous
import jax
import jax.numpy as jnp
from jax import lax
from jax.experimental import pallas as pl
from jax.experimental.pallas import tpu as pltpu
from jax.experimental.pallas import tpu_sc as plsc

N = 10000
E = 320000
H = 128
DE = 16

NP = 10240
NW = 32
EPW = 10240
EP = NW * EPW
NCHUNK = 80
G = 128
NB = NP // 128
SLICE = NP // 16

_f32 = jnp.float32
_i32 = jnp.int32


def _node_body(x_ref, w1_ref, b1_ref, w2_ref, b2_ref, gl_ref, as_ref, ad_ref,
               xw_ref, asrc_ref, adst_ref, ms_ref, md_ref):
    xb = x_ref[...]
    h = jnp.maximum(jnp.dot(xb, w1_ref[...], preferred_element_type=_f32) + b1_ref[...], 0.0)
    h = jnp.maximum(jnp.dot(h, w2_ref[...], preferred_element_type=_f32) + b2_ref[...], 0.0)
    xw = jnp.dot(h, gl_ref[...], preferred_element_type=_f32)
    asr = jnp.dot(xw, as_ref[...], preferred_element_type=_f32)
    adt = jnp.dot(xw, ad_ref[...], preferred_element_type=_f32)
    xw_ref[...] = xw
    asrc_ref[...] = asr
    adst_ref[...] = adt
    bs = jnp.max(asr, axis=(0, 1), keepdims=True)
    bd = jnp.max(adt, axis=(0, 1), keepdims=True)

    @pl.when(pl.program_id(0) == 0)
    def _():
        ms_ref[...] = bs
        md_ref[...] = bd

    @pl.when(pl.program_id(0) != 0)
    def _():
        ms_ref[...] = jnp.maximum(ms_ref[...], bs)
        md_ref[...] = jnp.maximum(md_ref[...], bd)


def _node_pass(x_p, n_w1, n_b1, n_w2, n_b2, gat_lin, att_src, att_dst):
    full = lambda s: pl.BlockSpec(s, lambda i: (0, 0))
    return pl.pallas_call(
        _node_body,
        grid=(NB,),
        in_specs=[
            pl.BlockSpec((128, H), lambda i: (i, 0)),
            full((H, H)), full((1, H)), full((H, H)), full((1, H)),
            full((H, H)), full((H, 1)), full((H, 1)),
        ],
        out_specs=[
            pl.BlockSpec((128, H), lambda i: (i, 0)),
            pl.BlockSpec((128, 1), lambda i: (i, 0)),
            pl.BlockSpec((128, 1), lambda i: (i, 0)),
            full((1, 1)), full((1, 1)),
        ],
        out_shape=[
            jax.ShapeDtypeStruct((NP, H), _f32),
            jax.ShapeDtypeStruct((NP, 1), _f32),
            jax.ShapeDtypeStruct((NP, 1), _f32),
            jax.ShapeDtypeStruct((1, 1), _f32),
            jax.ShapeDtypeStruct((1, 1), _f32),
        ],
    )(x_p, n_w1, n_b1.reshape(1, H), n_w2, n_b2.reshape(1, H),
      gat_lin, att_src.reshape(H, 1), att_dst.reshape(H, 1))


def _edge_body(eb_ref, w1_ref, b1_ref, w2_ref, b2_ref, gle_ref, ae_ref,
               ea_ref, s_ref, sm_ref):
    eb = eb_ref[...]
    e1 = jnp.maximum(jnp.dot(eb, w1_ref[...], preferred_element_type=_f32) + b1_ref[...], 0.0)
    e2 = jnp.maximum(jnp.dot(e1, w2_ref[...], preferred_element_type=_f32) + b2_ref[...], 0.0)
    v = jnp.dot(gle_ref[...], ae_ref[...], preferred_element_type=_f32)
    s = jnp.dot(e2, v, preferred_element_type=_f32)
    ea_ref[...] = e2
    s_ref[...] = s
    bm = jnp.max(s, axis=(0, 1), keepdims=True)

    @pl.when(pl.program_id(0) == 0)
    def _():
        sm_ref[...] = bm

    @pl.when(pl.program_id(0) != 0)
    def _():
        sm_ref[...] = jnp.maximum(sm_ref[...], bm)


def _edge_pass(ea_p, e_w1, e_b1, e_w2, e_b2, gat_lin_edge, att_edge):
    full = lambda s: pl.BlockSpec(s, lambda i: (0, 0))
    nblk = EP // 1024
    return pl.pallas_call(
        _edge_body,
        grid=(nblk,),
        in_specs=[
            pl.BlockSpec((1024, DE), lambda i: (i, 0)),
            full((DE, DE)), full((1, DE)), full((DE, DE)), full((1, DE)),
            full((DE, H)), full((H, 1)),
        ],
        out_specs=[
            pl.BlockSpec((1024, DE), lambda i: (i, 0)),
            pl.BlockSpec((1024, 1), lambda i: (i, 0)),
            full((1, 1)),
        ],
        out_shape=[
            jax.ShapeDtypeStruct((EP, DE), _f32),
            jax.ShapeDtypeStruct((EP, 1), _f32),
            jax.ShapeDtypeStruct((1, 1), _f32),
        ],
    )(ea_p, e_w1, e_b1.reshape(1, DE), e_w2, e_b2.reshape(1, DE),
      gat_lin_edge, att_edge.reshape(H, 1))


def _mid1_body(d0_ref, d1_ref, s0_ref, s1_ref, as_ref, ad_ref,
               ms_ref, md_ref, sm_ref, m_ref, wl_ref, sl_ref):
    deg = d0_ref[...] + d1_ref[...]
    ssum = s0_ref[...] + s1_ref[...]
    sloop = ssum / jnp.maximum(deg, 1.0)
    asr = as_ref[...]
    adt = ad_ref[...]
    tb = jnp.maximum(ms_ref[...] + sm_ref[...], asr + sloop) + adt
    m = jnp.where(tb >= 0, tb, 0.2 * tb)
    tl = asr + adt + sloop
    aloop = jnp.where(tl >= 0, tl, 0.2 * tl)
    m_ref[...] = m
    wl_ref[...] = jnp.exp(aloop - m)
    sl_ref[...] = sloop


def _mid1(deg_p, ssum_p, a_src, a_dst, ms, md, sm):
    full = lambda s: pl.BlockSpec(s, lambda i: (0, 0))
    col = lambda: pl.BlockSpec((128, 1), lambda i: (i, 0))
    d0 = deg_p[0].reshape(NP, 1)
    d1 = deg_p[1].reshape(NP, 1)
    s0 = ssum_p[0].reshape(NP, 1)
    s1 = ssum_p[1].reshape(NP, 1)
    return pl.pallas_call(
        _mid1_body,
        grid=(NB,),
        in_specs=[col(), col(), col(), col(), col(), col(),
                  full((1, 1)), full((1, 1)), full((1, 1))],
        out_specs=[col(), col(), col()],
        out_shape=[
            jax.ShapeDtypeStruct((NP, 1), _f32),
            jax.ShapeDtypeStruct((NP, 1), _f32),
            jax.ShapeDtypeStruct((NP, 1), _f32),
        ],
    )(d0, d1, s0, s1, a_src, a_dst, ms, md, sm)


def _mid2_body(m0_ref, m1_ref, xw_ref, wl_ref, dn0_ref, dn1_ref, gb_ref,
               w1_ref, w2_ref, out_ref, o1_ref, o2_ref):
    wl = wl_ref[...]
    den = dn0_ref[...] + dn1_ref[...] + wl + 1e-16
    out = (m0_ref[...] + m1_ref[...] + wl * xw_ref[...]) / den + gb_ref[...]
    out_ref[...] = out
    o1_ref[...] = jnp.dot(out, w1_ref[...], preferred_element_type=_f32)
    o2_ref[...] = jnp.dot(out, w2_ref[...], preferred_element_type=_f32)


def _mid2(msg_p, den_p, xw, wloop, gat_bias, eu_w1, eu_w2):
    full = lambda s: pl.BlockSpec(s, lambda i: (0, 0))
    row = lambda: pl.BlockSpec((128, H), lambda i: (i, 0))
    col = lambda: pl.BlockSpec((128, 1), lambda i: (i, 0))
    return pl.pallas_call(
        _mid2_body,
        grid=(NB,),
        in_specs=[row(), row(), row(), col(), col(), col(), full((1, H)),
                  full((H, H)), full((H, H))],
        out_specs=[row(), row(), row()],
        out_shape=[
            jax.ShapeDtypeStruct((NP, H), _f32),
            jax.ShapeDtypeStruct((NP, H), _f32),
            jax.ShapeDtypeStruct((NP, H), _f32),
        ],
    )(msg_p[0], msg_p[1], xw, wloop, den_p[0].reshape(NP, 1),
      den_p[1].reshape(NP, 1), gat_bias.reshape(1, H), eu_w1, eu_w2)


def _final_body(g_ref, ea_ref, w3_ref, b_ref, o_ref):
    o_ref[...] = (g_ref[...] +
                  jnp.dot(ea_ref[...], w3_ref[...], preferred_element_type=_f32) +
                  b_ref[...])


def _final(g, ea, eu_w3, eu_b):
    full = lambda s: pl.BlockSpec(s, lambda i: (0, 0))
    nblk = E // 512
    return pl.pallas_call(
        _final_body,
        grid=(nblk,),
        in_specs=[
            pl.BlockSpec((512, H), lambda i: (i, 0)),
            pl.BlockSpec((512, DE), lambda i: (i, 0)),
            full((DE, H)), full((1, H)),
        ],
        out_specs=pl.BlockSpec((512, H), lambda i: (i, 0)),
        out_shape=jax.ShapeDtypeStruct((E, H), _f32),
    )(g, ea, eu_w3, eu_b.reshape(1, H))


_MESH = plsc.VectorSubcoreMesh(core_axis_name="c", subcore_axis_name="s")


def _wid():
    return lax.axis_index("s") * 2 + lax.axis_index("c")


def _zero_1d(ref, nvec):
    z = jnp.zeros((16,), _f32)

    def body(i, _):
        ref[pl.ds(i * 16, 16)] = z
        return ()

    lax.fori_loop(0, nvec, body, ())


def _sc_a1_body(dst_hbm, s_hbm, deg_out, ssum_out,
                dstv, sv, onesv, zb, deg_sp, ssum_sp, sem):
    cid = lax.axis_index("c")
    sid = lax.axis_index("s")
    wid = _wid()
    _zero_1d(zb, SLICE // 16)
    pltpu.sync_copy(zb, deg_sp.at[pl.ds(sid * SLICE, SLICE)])
    pltpu.sync_copy(zb, ssum_sp.at[pl.ds(sid * SLICE, SLICE)])
    plsc.subcore_barrier()
    pltpu.sync_copy(dst_hbm.at[wid], dstv)
    pltpu.sync_copy(s_hbm.at[wid], sv)
    lane = jax.lax.iota(_i32, 16)

    def ones_body(c, _):
        base = wid * EPW + c * G
        for g in range(8):
            eid = base + g * 16 + lane
            onesv[c, pl.ds(g * 16, 16)] = jnp.where(eid < E, 1.0, 0.0)
        return ()

    lax.fori_loop(0, NCHUNK, ones_body, ())

    def scat_body(c, _):
        pltpu.sync_copy(onesv.at[c], deg_sp.at[dstv.at[c]], add=True)
        pltpu.sync_copy(sv.at[c], ssum_sp.at[dstv.at[c]], add=True)
        return ()

    lax.fori_loop(0, NCHUNK, scat_body, ())
    plsc.subcore_barrier()
    pltpu.sync_copy(deg_sp.at[pl.ds(sid * SLICE, SLICE)],
                    deg_out.at[cid, pl.ds(sid * SLICE, SLICE)])
    pltpu.sync_copy(ssum_sp.at[pl.ds(sid * SLICE, SLICE)],
                    ssum_out.at[cid, pl.ds(sid * SLICE, SLICE)])


def _sc_a1(dst3, s3):
    return pl.kernel(
        _sc_a1_body,
        out_type=[
            jax.ShapeDtypeStruct((2, NP), _f32),
            jax.ShapeDtypeStruct((2, NP), _f32),
        ],
        mesh=_MESH,
        scratch_types=[
            pltpu.VMEM((NCHUNK, G), _i32),
            pltpu.VMEM((NCHUNK, G), _f32),
            pltpu.VMEM((NCHUNK, G), _f32),
            pltpu.VMEM((SLICE,), _f32),
            pltpu.VMEM_SHARED((NP,), _f32),
            pltpu.VMEM_SHARED((NP,), _f32),
            pltpu.SemaphoreType.DMA,
        ],
    )(dst3, s3)


NPH = NP // 2
PSLICE = NPH // 16


def _sc_a2_body(src_hbm, dst_hbm, s_hbm, asrc_hbm, adst_hbm, m_hbm, xw_hbm,
                msg_out, den_out,
                srcv, dstv, sv, wtile, zb, idxb,
                rows0, ag0, bg0, mg0,
                rows1, ag1, bg1, mg1,
                msg_sp, den_sp, gsem0, gsem1, ssem, dsem):
    cid = lax.axis_index("c")
    sid = lax.axis_index("s")
    wid = _wid()
    bufs = ((rows0, ag0, bg0, mg0, gsem0), (rows1, ag1, bg1, mg1, gsem1))
    z = jnp.zeros((16,), _f32)
    lane = jax.lax.iota(_i32, 16)

    def zero_msg_slice():
        def zrow(i, _):
            for v in range(8):
                rows0[i, pl.ds(v * 16, 16)] = z
            return ()

        lax.fori_loop(0, G, zrow, ())
        for k in range(PSLICE // G):
            pltpu.sync_copy(rows0, msg_sp.at[pl.ds(sid * PSLICE + k * G, G)])
        pltpu.sync_copy(rows0.at[pl.ds(0, PSLICE % G)],
                        msg_sp.at[pl.ds(sid * PSLICE + (PSLICE // G) * G,
                                        PSLICE % G)])

    zero_msg_slice()
    _zero_1d(zb, SLICE // 16)
    pltpu.sync_copy(zb, den_sp.at[pl.ds(sid * SLICE, SLICE)])
    pltpu.sync_copy(src_hbm.at[wid], srcv)
    pltpu.sync_copy(dst_hbm.at[wid], dstv)
    pltpu.sync_copy(s_hbm.at[wid], sv)
    plsc.subcore_barrier()

    def fire(c, buf, ph):
        rws, ag, bg, mg, sem = buf
        pltpu.async_copy(xw_hbm.at[srcv.at[c]], rws, sem)
        if ph == 0:
            pltpu.async_copy(asrc_hbm.at[srcv.at[c]], ag, sem)
            pltpu.async_copy(adst_hbm.at[dstv.at[c]], bg, sem)
            pltpu.async_copy(m_hbm.at[dstv.at[c]], mg, sem)

    def drain(c, buf, ph):
        rws, ag, bg, mg, sem = buf
        pltpu.make_async_copy(xw_hbm.at[srcv.at[c]], rws, sem).wait()
        if ph == 0:
            pltpu.make_async_copy(asrc_hbm.at[srcv.at[c]], ag, sem).wait()
            pltpu.make_async_copy(adst_hbm.at[dstv.at[c]], bg, sem).wait()
            pltpu.make_async_copy(m_hbm.at[dstv.at[c]], mg, sem).wait()

    def make_phase(ph):
        nbase = ph * NPH

        def outer(i, _):
            for b in (0, 1):
                c = 2 * i + b
                buf = bufs[b]
                rws, ag, bg, mg, _sem = buf
                drain(c, buf, ph)

                @pl.when(c + 1 < NCHUNK)
                def _():
                    fire(c + 1, bufs[1 - b], ph)

                def group(g, _):
                    d16 = dstv[c, pl.ds(g * 16, 16)]
                    if ph == 0:
                        s16 = sv[c, pl.ds(g * 16, 16)]
                        av = ag[pl.ds(g * 16, 16)]
                        bv = bg[pl.ds(g * 16, 16)]
                        mv = mg[pl.ds(g * 16, 16)]
                        t = av + bv + s16
                        al = jnp.where(t >= 0, t, 0.2 * t)
                        eid = wid * EPW + c * G + g * 16 + lane
                        w16 = jnp.where(eid < E, jnp.exp(al - mv), 0.0)
                        wtile[c, pl.ds(g * 16, 16)] = w16
                    else:
                        w16 = wtile[c, pl.ds(g * 16, 16)]
                    dloc = d16 - nbase
                    inrange = (dloc >= 0) & (dloc < NPH)
                    idxb[pl.ds(g * 16, 16)] = jnp.where(inrange, dloc, NPH)
                    for e in range(16):
                        wv = w16.at[jnp.full((16,), e, _i32)].get(
                            mode="promise_in_bounds")
                        row = g * 16 + e
                        for v in range(8):
                            rws[row, pl.ds(v * 16, 16)] = (
                                rws[row, pl.ds(v * 16, 16)] * wv)
                    return ()

                lax.fori_loop(0, 8, group, ())
                if ph == 0:
                    pltpu.async_copy(wtile.at[c], den_sp.at[dstv.at[c]],
                                     dsem, add=True)
                pltpu.async_copy(rws, msg_sp.at[idxb], ssem, add=True).wait()
                if ph == 0:
                    pltpu.make_async_copy(wtile.at[c], den_sp.at[dstv.at[c]],
                                          dsem).wait()
            return ()

        fire(0, bufs[0], ph)
        lax.fori_loop(0, NCHUNK // 2, outer, ())
        plsc.subcore_barrier()
        pltpu.sync_copy(msg_sp.at[pl.ds(sid * PSLICE, PSLICE)],
                        msg_out.at[cid, pl.ds(nbase + sid * PSLICE, PSLICE)])

    make_phase(0)
    pltpu.sync_copy(den_sp.at[pl.ds(sid * SLICE, SLICE)],
                    den_out.at[cid, pl.ds(sid * SLICE, SLICE)])
    zero_msg_slice()
    plsc.subcore_barrier()
    make_phase(1)


def _sc_a2(src3, dst3, s3, a_src, a_dst, m_col, xw):
    return pl.kernel(
        _sc_a2_body,
        out_type=[
            jax.ShapeDtypeStruct((2, NP, H), _f32),
            jax.ShapeDtypeStruct((2, NP), _f32),
        ],
        mesh=_MESH,
        scratch_types=[
            pltpu.VMEM((NCHUNK, G), _i32),
            pltpu.VMEM((NCHUNK, G), _i32),
            pltpu.VMEM((NCHUNK, G), _f32),
            pltpu.VMEM((NCHUNK, G), _f32),
            pltpu.VMEM((SLICE,), _f32),
            pltpu.VMEM((G,), _i32),
            pltpu.VMEM((G, H), _f32),
            pltpu.VMEM((G,), _f32),
            pltpu.VMEM((G,), _f32),
            pltpu.VMEM((G,), _f32),
            pltpu.VMEM((G, H), _f32),
            pltpu.VMEM((G,), _f32),
            pltpu.VMEM((G,), _f32),
            pltpu.VMEM((G,), _f32),
            pltpu.VMEM_SHARED((NPH + 8, H), _f32),
            pltpu.VMEM_SHARED((NP,), _f32),
            pltpu.SemaphoreType.DMA,
            pltpu.SemaphoreType.DMA,
            pltpu.SemaphoreType.DMA,
            pltpu.SemaphoreType.DMA,
        ],
    )(src3, dst3, s3, a_src.reshape(NP), a_dst.reshape(NP), m_col.reshape(NP),
      xw)


def _sc_b_body(src_hbm, dst_hbm, o1_hbm, o2_hbm, g_out,
               srcv, dstv, r1a, r2a, r1b, r2b,
               gsem0, gsem1, wsem0, wsem1):
    wid = _wid()
    pltpu.sync_copy(src_hbm.at[wid], srcv)
    pltpu.sync_copy(dst_hbm.at[wid], dstv)
    bufs = ((r1a, r2a, gsem0, wsem0), (r1b, r2b, gsem1, wsem1))

    def out_slice(c):
        return g_out.at[pl.ds(wid * EPW + c * G, G)]

    def fire(c, buf):
        r1, r2, gsem, _w = buf
        pltpu.async_copy(o1_hbm.at[srcv.at[c]], r1, gsem)
        pltpu.async_copy(o2_hbm.at[dstv.at[c]], r2, gsem)

    def drain(c, buf):
        r1, r2, gsem, _w = buf
        pltpu.make_async_copy(o1_hbm.at[srcv.at[c]], r1, gsem).wait()
        pltpu.make_async_copy(o2_hbm.at[dstv.at[c]], r2, gsem).wait()

    fire(0, bufs[0])

    def outer(i, _):
        for b in (0, 1):
            c = 2 * i + b
            r1, r2, _g, wsem = bufs[b]

            @pl.when(c >= 2)
            def _():
                pltpu.make_async_copy(r1, out_slice(c - 2), wsem).wait()

            drain(c, bufs[b])

            @pl.when(c + 1 < NCHUNK)
            def _():
                fire(c + 1, bufs[1 - b])

            def row_add(r, _):
                for v in range(8):
                    r1[r, pl.ds(v * 16, 16)] = (
                        r1[r, pl.ds(v * 16, 16)] + r2[r, pl.ds(v * 16, 16)])
                return ()

            lax.fori_loop(0, G, row_add, ())
            pltpu.async_copy(r1, out_slice(c), wsem)
        return ()

    lax.fori_loop(0, NCHUNK // 2, outer, ())
    for last in (NCHUNK - 2, NCHUNK - 1):
        r1, _r2, _g, wsem = bufs[last % 2]
        pltpu.make_async_copy(r1, out_slice(last), wsem).wait()


def _sc_b(src3, dst3, out1, out2):
    return pl.kernel(
        _sc_b_body,
        out_type=jax.ShapeDtypeStruct((EP, H), _f32),
        mesh=_MESH,
        scratch_types=[
            pltpu.VMEM((NCHUNK, G), _i32),
            pltpu.VMEM((NCHUNK, G), _i32),
            pltpu.VMEM((G, H), _f32),
            pltpu.VMEM((G, H), _f32),
            pltpu.VMEM((G, H), _f32),
            pltpu.VMEM((G, H), _f32),
            pltpu.SemaphoreType.DMA,
            pltpu.SemaphoreType.DMA,
            pltpu.SemaphoreType.DMA,
            pltpu.SemaphoreType.DMA,
        ],
    )(src3, dst3, out1, out2)


def kernel(x, edge_index, edge_attr, n_w1, n_b1, n_w2, n_b2, e_w1, e_b1,
           e_w2, e_b2, gat_lin, gat_lin_edge, att_src, att_dst, att_edge,
           gat_bias, eu_w, eu_b):
    x_p = jnp.pad(x, ((0, NP - N), (0, 0)))
    ea_p = jnp.pad(edge_attr, ((0, EP - E), (0, 0)))
    src = edge_index[0].astype(_i32)
    dst = edge_index[1].astype(_i32)
    src3 = jnp.pad(src, (0, EP - E)).reshape(NW, NCHUNK, G)
    dst3 = jnp.pad(dst, (0, EP - E)).reshape(NW, NCHUNK, G)

    xw, a_src, a_dst, ms, md = _node_pass(
        x_p, n_w1, n_b1, n_w2, n_b2, gat_lin, att_src, att_dst)
    ea, s_col, sm = _edge_pass(
        ea_p, e_w1, e_b1, e_w2, e_b2, gat_lin_edge, att_edge)
    s3 = s_col.reshape(NW, NCHUNK, G)

    deg_p, ssum_p = _sc_a1(dst3, s3)
    m_col, wloop, _sloop = _mid1(deg_p, ssum_p, a_src, a_dst, ms, md, sm)
    msg_p, den_p = _sc_a2(src3, dst3, s3, a_src, a_dst, m_col, xw)
    out, out1, out2 = _mid2(msg_p, den_p, xw, wloop, gat_bias,
                            eu_w[:H], eu_w[H:2 * H])
    g = _sc_b(src3, dst3, out1, out2)
    new_ea = _final(g[:E], ea[:E], eu_w[2 * H:], eu_b)
    return (out[:N], new_ea)
